# Initial kernel scaffold; baseline (speedup 1.0000x reference)
import jax
import jax.numpy as jnp
from jax.experimental import pallas as pl
from jax.experimental.pallas import tpu as pltpu
from jax.experimental.pallas import tpu_sc as plsc

N_TOK = 16 * 576
N_EMB = 8192
D = 64
BLK = 512
NB = N_TOK // BLK
GW = 128


def _dist_argmin_kernel(x_ref, et_ref, inorm_ref, wnorm_ref, idx_ref, md_ref):
    x = x_ref[...]
    s = jax.lax.dot_general(
        x, et_ref[...], (((1,), (0,)), ((), ())),
        preferred_element_type=jnp.float32)
    dist = (inorm_ref[...] + wnorm_ref[...]) - 2.0 * s
    m = jnp.min(dist, axis=1, keepdims=True)
    iota = jax.lax.broadcasted_iota(jnp.int32, dist.shape, 1)
    idx = jnp.min(jnp.where(dist == m, iota, N_EMB), axis=1, keepdims=True)
    idx_ref[...] = idx
    md_ref[...] = m


def _sc_gather(embed, idx2):
    mesh = plsc.VectorSubcoreMesh(core_axis_name="core",
                                  subcore_axis_name="subcore")

    @pl.kernel(out_type=jax.ShapeDtypeStruct((N_TOK, D), jnp.float32),
               mesh=mesh)
    def gather_kernel(emb_hbm, i_hbm, o_hbm):
        def body(i_vmem, o_vmem):
            pltpu.sync_copy(emb_hbm.at[i_vmem.at[0]], o_vmem)

        pltpu.emit_pipeline(
            body,
            grid=(N_TOK // GW,),
            in_specs=[pl.BlockSpec((1, GW), index_map=lambda i: (0, i))],
            out_specs=[pl.BlockSpec((GW, D), index_map=lambda i: (i, 0))],
            core_axis_name=("core", "subcore"),
            dimension_semantics=(pltpu.PARALLEL,),
        )(i_hbm, o_hbm)

    return gather_kernel(embed, idx2)


def kernel(input, embed):
    dims = input.shape
    flat = input.reshape(-1, D)
    i_norm = jnp.sum(flat ** 2, axis=1).reshape(-1, 1)
    w_norm = jnp.sum(embed ** 2, axis=1).reshape(1, -1)
    et = embed.T

    idx, md = pl.pallas_call(
        _dist_argmin_kernel,
        grid=(NB,),
        in_specs=[
            pl.BlockSpec((BLK, D), lambda i: (i, 0)),
            pl.BlockSpec((D, N_EMB), lambda i: (0, 0)),
            pl.BlockSpec((BLK, 1), lambda i: (i, 0)),
            pl.BlockSpec((1, N_EMB), lambda i: (0, 0)),
        ],
        out_specs=[
            pl.BlockSpec((BLK, 1), lambda i: (i, 0)),
            pl.BlockSpec((BLK, 1), lambda i: (i, 0)),
        ],
        out_shape=[
            jax.ShapeDtypeStruct((N_TOK, 1), jnp.int32),
            jax.ShapeDtypeStruct((N_TOK, 1), jnp.float32),
        ],
        compiler_params=pltpu.CompilerParams(
            dimension_semantics=("parallel",)),
    )(flat, et, i_norm, w_norm)

    embs = _sc_gather(embed, idx.reshape(1, N_TOK))
    loss = jnp.sum(md) * (1.25 / (N_TOK * D))
    return embs.reshape(dims[0], dims[1], -1), loss


# speedup vs baseline: 1.4127x; 1.4127x over previous
import jax
import jax.numpy as jnp
from jax.experimental import pallas as pl
from jax.experimental.pallas import tpu as pltpu
from jax.experimental.pallas import tpu_sc as plsc

N_TOK = 16 * 576
N_EMB = 8192
D = 64
BLK = 512
NB = N_TOK // BLK
GW = 128
DP = 128


def _dist_argmin_kernel(x_ref, et_ref, inorm_ref, wnorm_ref, idx_ref, md_ref):
    x = x_ref[...]
    s = jax.lax.dot_general(
        x, et_ref[...], (((1,), (0,)), ((), ())),
        preferred_element_type=jnp.float32)
    dist = (inorm_ref[...] + wnorm_ref[...]) - 2.0 * s
    m = jnp.min(dist, axis=1, keepdims=True)
    iota = jax.lax.broadcasted_iota(jnp.int32, dist.shape, 1)
    idx = jnp.min(jnp.where(dist == m, iota, N_EMB), axis=1, keepdims=True)
    idx_ref[...] = idx
    md_ref[...] = m


def _sc_gather(embed_p, idx2):
    mesh = plsc.VectorSubcoreMesh(core_axis_name="core",
                                  subcore_axis_name="subcore")

    @pl.kernel(out_type=jax.ShapeDtypeStruct((N_TOK, DP), jnp.float32),
               mesh=mesh)
    def gather_kernel(emb_hbm, i_hbm, o_hbm):
        def body(i_vmem, o_vmem):
            pltpu.sync_copy(emb_hbm.at[i_vmem.at[0]], o_vmem)

        pltpu.emit_pipeline(
            body,
            grid=(N_TOK // GW,),
            in_specs=[pl.BlockSpec((1, GW), index_map=lambda i: (0, i))],
            out_specs=[pl.BlockSpec((GW, DP), index_map=lambda i: (i, 0))],
            core_axis_name=("core", "subcore"),
            dimension_semantics=(pltpu.PARALLEL,),
        )(i_hbm, o_hbm)

    return gather_kernel(embed_p, idx2)


def kernel(input, embed):
    dims = input.shape
    flat = input.reshape(-1, D)
    i_norm = jnp.sum(flat ** 2, axis=1).reshape(-1, 1)
    w_norm = jnp.sum(embed ** 2, axis=1).reshape(1, -1)
    et = embed.T

    idx, md = pl.pallas_call(
        _dist_argmin_kernel,
        grid=(NB,),
        in_specs=[
            pl.BlockSpec((BLK, D), lambda i: (i, 0)),
            pl.BlockSpec((D, N_EMB), lambda i: (0, 0)),
            pl.BlockSpec((BLK, 1), lambda i: (i, 0)),
            pl.BlockSpec((1, N_EMB), lambda i: (0, 0)),
        ],
        out_specs=[
            pl.BlockSpec((BLK, 1), lambda i: (i, 0)),
            pl.BlockSpec((BLK, 1), lambda i: (i, 0)),
        ],
        out_shape=[
            jax.ShapeDtypeStruct((N_TOK, 1), jnp.int32),
            jax.ShapeDtypeStruct((N_TOK, 1), jnp.float32),
        ],
        compiler_params=pltpu.CompilerParams(
            dimension_semantics=("parallel",)),
    )(flat, et, i_norm, w_norm)

    embed_p = jnp.pad(embed, ((0, 0), (0, DP - D)))
    embs = _sc_gather(embed_p, idx.reshape(1, N_TOK))[:, :D]
    loss = jnp.sum(md) * (1.25 / (N_TOK * D))
    return embs.reshape(dims[0], dims[1], -1), loss


# speedup vs baseline: 1.4996x; 1.0615x over previous
import jax
import jax.numpy as jnp
from jax.experimental import pallas as pl
from jax.experimental.pallas import tpu as pltpu
from jax.experimental.pallas import tpu_sc as plsc

N_TOK = 16 * 576
N_EMB = 8192
D = 64
BLK = 512
NB = N_TOK // BLK
GW = 128
DP = 128


W = 1024
NCH = N_EMB // W


def _dist_argmin_kernel(x_ref, et2_ref, inorm_ref, wnorm_ref, idx_ref, md_ref):
    x = x_ref[...]
    i_col = inorm_ref[...]
    bv = bi = None
    for c in range(NCH):
        sl = pl.ds(c * W, W)
        s = jax.lax.dot_general(
            x, et2_ref[:, sl], (((1,), (0,)), ((), ())),
            preferred_element_type=jnp.float32)
        d = (i_col + wnorm_ref[:, sl]) + s
        ii = jax.lax.broadcasted_iota(jnp.int32, (BLK, W), 1) + (c * W)
        if bv is None:
            bv, bi = d, ii
        else:
            take = d < bv
            bi = jnp.where(take, ii, bi)
            bv = jnp.minimum(d, bv)
    m = jnp.min(bv, axis=1, keepdims=True)
    idx = jnp.min(jnp.where(bv == m, bi, N_EMB), axis=1, keepdims=True)
    idx_ref[...] = idx
    md_ref[...] = m


def _sc_gather(embed_p, idx2):
    mesh = plsc.VectorSubcoreMesh(core_axis_name="core",
                                  subcore_axis_name="subcore")

    @pl.kernel(out_type=jax.ShapeDtypeStruct((N_TOK, DP), jnp.float32),
               mesh=mesh)
    def gather_kernel(emb_hbm, i_hbm, o_hbm):
        def body(i_vmem, o_vmem):
            pltpu.sync_copy(emb_hbm.at[i_vmem.at[0]], o_vmem)

        pltpu.emit_pipeline(
            body,
            grid=(N_TOK // GW,),
            in_specs=[pl.BlockSpec((1, GW), index_map=lambda i: (0, i))],
            out_specs=[pl.BlockSpec((GW, DP), index_map=lambda i: (i, 0))],
            core_axis_name=("core", "subcore"),
            dimension_semantics=(pltpu.PARALLEL,),
        )(i_hbm, o_hbm)

    return gather_kernel(embed_p, idx2)


def kernel(input, embed):
    dims = input.shape
    flat = input.reshape(-1, D)
    i_norm = jnp.sum(flat ** 2, axis=1).reshape(-1, 1)
    w_norm = jnp.sum(embed ** 2, axis=1).reshape(1, -1)
    et2 = embed.T * (-2.0)

    idx, md = pl.pallas_call(
        _dist_argmin_kernel,
        grid=(NB,),
        in_specs=[
            pl.BlockSpec((BLK, D), lambda i: (i, 0)),
            pl.BlockSpec((D, N_EMB), lambda i: (0, 0)),
            pl.BlockSpec((BLK, 1), lambda i: (i, 0)),
            pl.BlockSpec((1, N_EMB), lambda i: (0, 0)),
        ],
        out_specs=[
            pl.BlockSpec((BLK, 1), lambda i: (i, 0)),
            pl.BlockSpec((BLK, 1), lambda i: (i, 0)),
        ],
        out_shape=[
            jax.ShapeDtypeStruct((N_TOK, 1), jnp.int32),
            jax.ShapeDtypeStruct((N_TOK, 1), jnp.float32),
        ],
        compiler_params=pltpu.CompilerParams(
            dimension_semantics=("parallel",)),
    )(flat, et2, i_norm, w_norm)

    embed_p = jnp.pad(embed, ((0, 0), (0, DP - D)))
    embs = _sc_gather(embed_p, idx.reshape(1, N_TOK))[:, :D]
    loss = jnp.sum(md) * (1.25 / (N_TOK * D))
    return embs.reshape(dims[0], dims[1], -1), loss


# speedup vs baseline: 1.5337x; 1.0227x over previous
import jax
import jax.numpy as jnp
from jax.experimental import pallas as pl
from jax.experimental.pallas import tpu as pltpu
from jax.experimental.pallas import tpu_sc as plsc

N_TOK = 16 * 576
N_EMB = 8192
D = 64
BLK = 1024
NB = N_TOK // BLK
GW = 128
DP = 128


W = 1024
NCH = N_EMB // W


def _dist_argmin_kernel(x_ref, et2_ref, inorm_ref, wnorm_ref, idx_ref, md_ref):
    x = x_ref[...]
    i_col = inorm_ref[...]
    bv = bi = None
    for c in range(NCH):
        sl = pl.ds(c * W, W)
        s = jax.lax.dot_general(
            x, et2_ref[:, sl], (((1,), (0,)), ((), ())),
            preferred_element_type=jnp.float32)
        d = (i_col + wnorm_ref[:, sl]) + s
        ii = jax.lax.broadcasted_iota(jnp.int32, (BLK, W), 1) + (c * W)
        if bv is None:
            bv, bi = d, ii
        else:
            take = d < bv
            bi = jnp.where(take, ii, bi)
            bv = jnp.minimum(d, bv)
    m = jnp.min(bv, axis=1, keepdims=True)
    idx = jnp.min(jnp.where(bv == m, bi, N_EMB), axis=1, keepdims=True)
    idx_ref[...] = idx
    md_ref[...] = m


def _sc_gather(embed_p, idx2):
    mesh = plsc.VectorSubcoreMesh(core_axis_name="core",
                                  subcore_axis_name="subcore")

    @pl.kernel(out_type=jax.ShapeDtypeStruct((N_TOK, DP), jnp.float32),
               mesh=mesh)
    def gather_kernel(emb_hbm, i_hbm, o_hbm):
        def body(i_vmem, o_vmem):
            pltpu.sync_copy(emb_hbm.at[i_vmem.at[0]], o_vmem)

        pltpu.emit_pipeline(
            body,
            grid=(N_TOK // GW,),
            in_specs=[pl.BlockSpec((1, GW), index_map=lambda i: (0, i))],
            out_specs=[pl.BlockSpec((GW, DP), index_map=lambda i: (i, 0))],
            core_axis_name=("core", "subcore"),
            dimension_semantics=(pltpu.PARALLEL,),
        )(i_hbm, o_hbm)

    return gather_kernel(embed_p, idx2)


def kernel(input, embed):
    dims = input.shape
    flat = input.reshape(-1, D)
    i_norm = jnp.sum(flat ** 2, axis=1).reshape(-1, 1)
    w_norm = jnp.sum(embed ** 2, axis=1).reshape(1, -1)
    et2 = embed.T * (-2.0)

    idx, md = pl.pallas_call(
        _dist_argmin_kernel,
        grid=(NB,),
        in_specs=[
            pl.BlockSpec((BLK, D), lambda i: (i, 0)),
            pl.BlockSpec((D, N_EMB), lambda i: (0, 0)),
            pl.BlockSpec((BLK, 1), lambda i: (i, 0)),
            pl.BlockSpec((1, N_EMB), lambda i: (0, 0)),
        ],
        out_specs=[
            pl.BlockSpec((BLK, 1), lambda i: (i, 0)),
            pl.BlockSpec((BLK, 1), lambda i: (i, 0)),
        ],
        out_shape=[
            jax.ShapeDtypeStruct((N_TOK, 1), jnp.int32),
            jax.ShapeDtypeStruct((N_TOK, 1), jnp.float32),
        ],
        compiler_params=pltpu.CompilerParams(
            dimension_semantics=("parallel",)),
    )(flat, et2, i_norm, w_norm)

    embed_p = jnp.pad(embed, ((0, 0), (0, DP - D)))
    embs = _sc_gather(embed_p, idx.reshape(1, N_TOK))[:, :D]
    loss = jnp.sum(md) * (1.25 / (N_TOK * D))
    return embs.reshape(dims[0], dims[1], -1), loss
